# Initial kernel scaffold; baseline (speedup 1.0000x reference)
import jax
import jax.numpy as jnp
from jax.experimental import pallas as pl


def kernel(gate_type_idx, gate_arity, gate_index_norm, gate_is_directional, qubit_degree_norm, edge_src_gate, edge_dst_qubit, emb_table, W_gate_in, b_gate_in, W_qubit_in, b_qubit_in, Wl_gq_0, bl_gq_0, Wr_gq_0, Wl_qg_0, bl_qg_0, Wr_qg_0, ln_g_0_s, ln_g_0_b, ln_q_0_s, ln_q_0_b, Wl_gq_1, bl_gq_1, Wr_gq_1, Wl_qg_1, bl_qg_1, Wr_qg_1, ln_g_1_s, ln_g_1_b, ln_q_1_s, ln_q_1_b, Wl_gq_2, bl_gq_2, Wr_gq_2, Wl_qg_2, bl_qg_2, Wr_qg_2, ln_g_2_s, ln_g_2_b, ln_q_2_s, ln_q_2_b, Wl_gq_3, bl_gq_3, Wr_gq_3, Wl_qg_3, bl_qg_3, Wr_qg_3, ln_g_3_s, ln_g_3_b, ln_q_3_s, ln_q_3_b, W_head1, b_head1, W_head2, b_head2):
    raise NotImplementedError("write your pallas kernel here")


# speedup vs baseline: 1.0435x; 1.0435x over previous
import functools

import jax
import jax.numpy as jnp
from jax import lax
from jax.experimental import pallas as pl

NUM_GATE_TYPES = 30
MASK_TOKEN_IDX = NUM_GATE_TYPES + 1
HIDDEN = 256
NUM_LAYERS = 4
GATE_EMB_DIM = 16
MASK_RATIO = 0.15
QUBIT_MASK_RATIO = 0.15
N_GATES = 100000
N_QUBITS = 10000
N_EDGES = 200000

GB = 2000
QB = 2000


def _silu(x):
    return x / (1.0 + jnp.exp(-x))


def _ln(x, s, b):
    mu = jnp.mean(x, axis=-1, keepdims=True)
    xc = x - mu
    var = jnp.mean(xc * xc, axis=-1, keepdims=True)
    return xc * jax.lax.rsqrt(var + 1e-5) * s + b


def _input_body(mtype_ref, arity_ref, isdir_ref, posf_ref, emb_ref, w16_ref,
                wadp_ref, b_ref, out_ref):
    mtype = mtype_ref[...]
    oh = (mtype == lax.broadcasted_iota(jnp.int32, (1, NUM_GATE_TYPES + 2), 1)
          ).astype(jnp.float32)
    M = lax.dot_general(emb_ref[...], w16_ref[...], (((1,), (1,)), ((), ())),
                        preferred_element_type=jnp.float32)
    x = lax.dot_general(oh, M, (((1,), (0,)), ((), ())),
                        preferred_element_type=jnp.float32)
    wadp = wadp_ref[...]
    x = x + arity_ref[...] * wadp[0:1, :]
    x = x + isdir_ref[...] * wadp[1:2, :]
    x = x + posf_ref[...] * wadp[2:3, :]
    out_ref[...] = x + b_ref[...]


def _gate_input_proj(mtype, arity, isdir, posf, emb_table, W_gate_in, b_gate_in):
    n = N_GATES
    w16 = W_gate_in[:, :GATE_EMB_DIM]
    wadp = jnp.transpose(W_gate_in[:, GATE_EMB_DIM:])
    grid = (n // GB,)
    col = lambda i: (i, 0)
    return pl.pallas_call(
        _input_body,
        grid=grid,
        in_specs=[
            pl.BlockSpec((GB, 1), col),
            pl.BlockSpec((GB, 1), col),
            pl.BlockSpec((GB, 1), col),
            pl.BlockSpec((GB, 1), col),
            pl.BlockSpec((NUM_GATE_TYPES + 2, GATE_EMB_DIM), lambda i: (0, 0)),
            pl.BlockSpec((HIDDEN, GATE_EMB_DIM), lambda i: (0, 0)),
            pl.BlockSpec((3, HIDDEN), lambda i: (0, 0)),
            pl.BlockSpec((1, HIDDEN), lambda i: (0, 0)),
        ],
        out_specs=pl.BlockSpec((GB, HIDDEN), col),
        out_shape=jax.ShapeDtypeStruct((n, HIDDEN), jnp.float32),
    )(mtype.reshape(n, 1), arity.reshape(n, 1), isdir.reshape(n, 1),
      posf.reshape(n, 1), emb_table, w16, wadp, b_gate_in.reshape(1, HIDDEN))


def _qubit_input_body(deg_ref, w_ref, b_ref, out_ref):
    out_ref[...] = deg_ref[...] * w_ref[...] + b_ref[...]


def _qubit_input_proj(masked_deg, W_qubit_in, b_qubit_in):
    n = N_QUBITS
    return pl.pallas_call(
        _qubit_input_body,
        grid=(n // QB,),
        in_specs=[
            pl.BlockSpec((QB, 1), lambda i: (i, 0)),
            pl.BlockSpec((1, HIDDEN), lambda i: (0, 0)),
            pl.BlockSpec((1, HIDDEN), lambda i: (0, 0)),
        ],
        out_specs=pl.BlockSpec((QB, HIDDEN), lambda i: (i, 0)),
        out_shape=jax.ShapeDtypeStruct((n, HIDDEN), jnp.float32),
    )(masked_deg.reshape(n, 1), W_qubit_in.reshape(1, HIDDEN),
      b_qubit_in.reshape(1, HIDDEN))


def _dense_body(agg_ref, deg_ref, x_ref, wl_ref, bl_ref, wr_ref, s_ref, b_ref,
                out_ref):
    x = x_ref[...]
    agg = agg_ref[...] / deg_ref[...]
    h = lax.dot_general(agg, wl_ref[...], (((1,), (1,)), ((), ())),
                        preferred_element_type=jnp.float32)
    h = h + lax.dot_general(x, wr_ref[...], (((1,), (1,)), ((), ())),
                            preferred_element_type=jnp.float32)
    h = h + bl_ref[...]
    out_ref[...] = _ln(_silu(h), s_ref[...], b_ref[...]) + x


def _dense_update(agg, deg, x, Wl, bl, Wr, ln_s, ln_b, blk):
    n = x.shape[0]
    full = lambda i: (0, 0)
    col = lambda i: (i, 0)
    return pl.pallas_call(
        _dense_body,
        grid=(n // blk,),
        in_specs=[
            pl.BlockSpec((blk, HIDDEN), col),
            pl.BlockSpec((blk, 1), col),
            pl.BlockSpec((blk, HIDDEN), col),
            pl.BlockSpec((HIDDEN, HIDDEN), full),
            pl.BlockSpec((1, HIDDEN), full),
            pl.BlockSpec((HIDDEN, HIDDEN), full),
            pl.BlockSpec((1, HIDDEN), full),
            pl.BlockSpec((1, HIDDEN), full),
        ],
        out_specs=pl.BlockSpec((blk, HIDDEN), col),
        out_shape=jax.ShapeDtypeStruct((n, HIDDEN), jnp.float32),
    )(agg, deg.reshape(n, 1), x, Wl, bl.reshape(1, HIDDEN), Wr,
      ln_s.reshape(1, HIDDEN), ln_b.reshape(1, HIDDEN))


def _head_body(x_ref, w1_ref, b1_ref, w2_ref, b2_ref, out_ref):
    h = lax.dot_general(x_ref[...], w1_ref[...], (((1,), (1,)), ((), ())),
                        preferred_element_type=jnp.float32)
    h = _silu(h + b1_ref[...])
    out_ref[...] = lax.dot_general(h, w2_ref[...], (((1,), (1,)), ((), ())),
                                   preferred_element_type=jnp.float32) + b2_ref[...]


def _head(gate_x, W_head1, b_head1, W_head2, b_head2):
    n = N_GATES
    nt = NUM_GATE_TYPES + 1
    full = lambda i: (0, 0)
    return pl.pallas_call(
        _head_body,
        grid=(n // GB,),
        in_specs=[
            pl.BlockSpec((GB, HIDDEN), lambda i: (i, 0)),
            pl.BlockSpec((HIDDEN, HIDDEN), full),
            pl.BlockSpec((1, HIDDEN), full),
            pl.BlockSpec((nt, HIDDEN), full),
            pl.BlockSpec((1, nt), full),
        ],
        out_specs=pl.BlockSpec((GB, nt), lambda i: (i, 0)),
        out_shape=jax.ShapeDtypeStruct((n, nt), jnp.float32),
    )(gate_x, W_head1, b_head1.reshape(1, HIDDEN), W_head2,
      b_head2.reshape(1, nt))


def _segment_mean_aggs(gate_x, qubit_x, src_g, dst_q, deg_q, deg_g):
    agg_q = jax.ops.segment_sum(jnp.take(gate_x, src_g, axis=0), dst_q,
                                num_segments=N_QUBITS)
    agg_g = jax.ops.segment_sum(jnp.take(qubit_x, dst_q, axis=0), src_g,
                                num_segments=N_GATES)
    return agg_q, agg_g


def kernel(gate_type_idx, gate_arity, gate_index_norm, gate_is_directional, qubit_degree_norm, edge_src_gate, edge_dst_qubit, emb_table, W_gate_in, b_gate_in, W_qubit_in, b_qubit_in, Wl_gq_0, bl_gq_0, Wr_gq_0, Wl_qg_0, bl_qg_0, Wr_qg_0, ln_g_0_s, ln_g_0_b, ln_q_0_s, ln_q_0_b, Wl_gq_1, bl_gq_1, Wr_gq_1, Wl_qg_1, bl_qg_1, Wr_qg_1, ln_g_1_s, ln_g_1_b, ln_q_1_s, ln_q_1_b, Wl_gq_2, bl_gq_2, Wr_gq_2, Wl_qg_2, bl_qg_2, Wr_qg_2, ln_g_2_s, ln_g_2_b, ln_q_2_s, ln_q_2_b, Wl_gq_3, bl_gq_3, Wr_gq_3, Wl_qg_3, bl_qg_3, Wr_qg_3, ln_g_3_s, ln_g_3_b, ln_q_3_s, ln_q_3_b, W_head1, b_head1, W_head2, b_head2):
    Wl = [Wl_gq_0, Wl_qg_0, Wl_gq_1, Wl_qg_1, Wl_gq_2, Wl_qg_2, Wl_gq_3, Wl_qg_3]
    bl = [bl_gq_0, bl_qg_0, bl_gq_1, bl_qg_1, bl_gq_2, bl_qg_2, bl_gq_3, bl_qg_3]
    Wr = [Wr_gq_0, Wr_qg_0, Wr_gq_1, Wr_qg_1, Wr_gq_2, Wr_qg_2, Wr_gq_3, Wr_qg_3]
    ln_s = [ln_g_0_s, ln_q_0_s, ln_g_1_s, ln_q_1_s, ln_g_2_s, ln_q_2_s,
            ln_g_3_s, ln_q_3_s]
    ln_b = [ln_g_0_b, ln_q_0_b, ln_g_1_b, ln_q_1_b, ln_g_2_b, ln_q_2_b,
            ln_g_3_b, ln_q_3_b]

    k1, k2, k3, k4 = jax.random.split(jax.random.key(42), 4)
    rand = jax.random.uniform(k1, (N_GATES,))
    mask_pos = rand < MASK_RATIO
    mask_rand = jax.random.uniform(k2, (N_GATES,))
    masked_type = jnp.where(mask_pos & (mask_rand < 0.8), MASK_TOKEN_IDX,
                            gate_type_idx)
    rnd_gates = jax.random.randint(k3, (N_GATES,), 0, NUM_GATE_TYPES)
    masked_type = jnp.where(mask_pos & (mask_rand >= 0.8) & (mask_rand < 0.9),
                            rnd_gates, masked_type)
    masked_arity = jnp.where(mask_pos, 0.0, gate_arity.astype(jnp.float32))
    masked_posf = jnp.where(mask_pos, 0.0, gate_index_norm)
    q_mask = jax.random.uniform(k4, (N_QUBITS,)) < QUBIT_MASK_RATIO
    masked_deg = jnp.where(q_mask, 0.0, qubit_degree_norm)

    gate_x = _gate_input_proj(masked_type.astype(jnp.int32), masked_arity,
                              gate_is_directional.astype(jnp.float32),
                              masked_posf, emb_table, W_gate_in, b_gate_in)
    qubit_x = _qubit_input_proj(masked_deg, W_qubit_in, b_qubit_in)

    ones = jnp.ones((N_EDGES,), jnp.float32)
    deg_q = jnp.maximum(jax.ops.segment_sum(ones, edge_dst_qubit,
                                            num_segments=N_QUBITS), 1.0)
    deg_g = jnp.maximum(jax.ops.segment_sum(ones, edge_src_gate,
                                            num_segments=N_GATES), 1.0)

    for l in range(NUM_LAYERS):
        agg_q, agg_g = _segment_mean_aggs(gate_x, qubit_x, edge_src_gate,
                                          edge_dst_qubit, deg_q, deg_g)
        new_q = _dense_update(agg_q, deg_q, qubit_x, Wl[2 * l], bl[2 * l],
                              Wr[2 * l], ln_s[2 * l + 1], ln_b[2 * l + 1], QB)
        new_g = _dense_update(agg_g, deg_g, gate_x, Wl[2 * l + 1], bl[2 * l + 1],
                              Wr[2 * l + 1], ln_s[2 * l], ln_b[2 * l], GB)
        gate_x, qubit_x = new_g, new_q

    return _head(gate_x, W_head1, b_head1, W_head2, b_head2)


# speedup vs baseline: 1.3311x; 1.2756x over previous
import functools

import jax
import jax.numpy as jnp
from jax import lax
from jax.experimental import pallas as pl
from jax.experimental.pallas import tpu as pltpu
from jax.experimental.pallas import tpu_sc as plsc

NUM_GATE_TYPES = 30
MASK_TOKEN_IDX = NUM_GATE_TYPES + 1
HIDDEN = 256
NUM_LAYERS = 4
GATE_EMB_DIM = 16
MASK_RATIO = 0.15
QUBIT_MASK_RATIO = 0.15
N_GATES = 100000
N_QUBITS = 10000
N_EDGES = 200000

GB = 2000
QB = 2000


def _silu(x):
    return x / (1.0 + jnp.exp(-x))


def _ln(x, s, b):
    mu = jnp.mean(x, axis=-1, keepdims=True)
    xc = x - mu
    var = jnp.mean(xc * xc, axis=-1, keepdims=True)
    return xc * jax.lax.rsqrt(var + 1e-5) * s + b


def _input_body(mtype_ref, arity_ref, isdir_ref, posf_ref, emb_ref, w16_ref,
                wadp_ref, b_ref, out_ref):
    mtype = mtype_ref[...]
    oh = (mtype == lax.broadcasted_iota(jnp.int32, (1, NUM_GATE_TYPES + 2), 1)
          ).astype(jnp.float32)
    M = lax.dot_general(emb_ref[...], w16_ref[...], (((1,), (1,)), ((), ())),
                        preferred_element_type=jnp.float32)
    x = lax.dot_general(oh, M, (((1,), (0,)), ((), ())),
                        preferred_element_type=jnp.float32)
    wadp = wadp_ref[...]
    x = x + arity_ref[...] * wadp[0:1, :]
    x = x + isdir_ref[...] * wadp[1:2, :]
    x = x + posf_ref[...] * wadp[2:3, :]
    out_ref[...] = x + b_ref[...]


def _gate_input_proj(mtype, arity, isdir, posf, emb_table, W_gate_in, b_gate_in):
    n = N_GATES
    w16 = W_gate_in[:, :GATE_EMB_DIM]
    wadp = jnp.transpose(W_gate_in[:, GATE_EMB_DIM:])
    grid = (n // GB,)
    col = lambda i: (i, 0)
    return pl.pallas_call(
        _input_body,
        grid=grid,
        in_specs=[
            pl.BlockSpec((GB, 1), col),
            pl.BlockSpec((GB, 1), col),
            pl.BlockSpec((GB, 1), col),
            pl.BlockSpec((GB, 1), col),
            pl.BlockSpec((NUM_GATE_TYPES + 2, GATE_EMB_DIM), lambda i: (0, 0)),
            pl.BlockSpec((HIDDEN, GATE_EMB_DIM), lambda i: (0, 0)),
            pl.BlockSpec((3, HIDDEN), lambda i: (0, 0)),
            pl.BlockSpec((1, HIDDEN), lambda i: (0, 0)),
        ],
        out_specs=pl.BlockSpec((GB, HIDDEN), col),
        out_shape=jax.ShapeDtypeStruct((n, HIDDEN), jnp.float32),
    )(mtype.reshape(n, 1), arity.reshape(n, 1), isdir.reshape(n, 1),
      posf.reshape(n, 1), emb_table, w16, wadp, b_gate_in.reshape(1, HIDDEN))


def _qubit_input_body(deg_ref, w_ref, b_ref, out_ref):
    out_ref[...] = deg_ref[...] * w_ref[...] + b_ref[...]


def _qubit_input_proj(masked_deg, W_qubit_in, b_qubit_in):
    n = N_QUBITS
    return pl.pallas_call(
        _qubit_input_body,
        grid=(n // QB,),
        in_specs=[
            pl.BlockSpec((QB, 1), lambda i: (i, 0)),
            pl.BlockSpec((1, HIDDEN), lambda i: (0, 0)),
            pl.BlockSpec((1, HIDDEN), lambda i: (0, 0)),
        ],
        out_specs=pl.BlockSpec((QB, HIDDEN), lambda i: (i, 0)),
        out_shape=jax.ShapeDtypeStruct((n, HIDDEN), jnp.float32),
    )(masked_deg.reshape(n, 1), W_qubit_in.reshape(1, HIDDEN),
      b_qubit_in.reshape(1, HIDDEN))


def _dense_body(agg_ref, deg_ref, x_ref, wl_ref, bl_ref, wr_ref, s_ref, b_ref,
                out_ref):
    x = x_ref[...]
    agg = agg_ref[...] / jnp.maximum(deg_ref[...], 1.0)
    h = lax.dot_general(agg, wl_ref[...], (((1,), (1,)), ((), ())),
                        preferred_element_type=jnp.float32)
    h = h + lax.dot_general(x, wr_ref[...], (((1,), (1,)), ((), ())),
                            preferred_element_type=jnp.float32)
    h = h + bl_ref[...]
    out_ref[...] = _ln(_silu(h), s_ref[...], b_ref[...]) + x


def _dense_update(agg, deg, x, Wl, bl, Wr, ln_s, ln_b, blk):
    n = x.shape[0]
    full = lambda i: (0, 0)
    col = lambda i: (i, 0)
    return pl.pallas_call(
        _dense_body,
        grid=(n // blk,),
        in_specs=[
            pl.BlockSpec((blk, HIDDEN), col),
            pl.BlockSpec((blk, 1), col),
            pl.BlockSpec((blk, HIDDEN), col),
            pl.BlockSpec((HIDDEN, HIDDEN), full),
            pl.BlockSpec((1, HIDDEN), full),
            pl.BlockSpec((HIDDEN, HIDDEN), full),
            pl.BlockSpec((1, HIDDEN), full),
            pl.BlockSpec((1, HIDDEN), full),
        ],
        out_specs=pl.BlockSpec((blk, HIDDEN), col),
        out_shape=jax.ShapeDtypeStruct((n, HIDDEN), jnp.float32),
    )(agg, deg.reshape(n, 1), x, Wl, bl.reshape(1, HIDDEN), Wr,
      ln_s.reshape(1, HIDDEN), ln_b.reshape(1, HIDDEN))


def _head_body(x_ref, w1_ref, b1_ref, w2_ref, b2_ref, out_ref):
    h = lax.dot_general(x_ref[...], w1_ref[...], (((1,), (1,)), ((), ())),
                        preferred_element_type=jnp.float32)
    h = _silu(h + b1_ref[...])
    out_ref[...] = lax.dot_general(h, w2_ref[...], (((1,), (1,)), ((), ())),
                                   preferred_element_type=jnp.float32) + b2_ref[...]


def _head(gate_x, W_head1, b_head1, W_head2, b_head2):
    n = N_GATES
    nt = NUM_GATE_TYPES + 1
    full = lambda i: (0, 0)
    return pl.pallas_call(
        _head_body,
        grid=(n // GB,),
        in_specs=[
            pl.BlockSpec((GB, HIDDEN), lambda i: (i, 0)),
            pl.BlockSpec((HIDDEN, HIDDEN), full),
            pl.BlockSpec((1, HIDDEN), full),
            pl.BlockSpec((nt, HIDDEN), full),
            pl.BlockSpec((1, nt), full),
        ],
        out_specs=pl.BlockSpec((GB, nt), lambda i: (i, 0)),
        out_shape=jax.ShapeDtypeStruct((n, nt), jnp.float32),
    )(gate_x, W_head1, b_head1.reshape(1, HIDDEN), W_head2,
      b_head2.reshape(1, nt))


_NC, _NS = 2, 16
_NW = _NC * _NS
_C = 128
_R = 200
_RZ = 208
E_PAD = N_EDGES + 256


def _sc_segsum(NR, D, gather):
    mesh = plsc.VectorSubcoreMesh(core_axis_name="c", subcore_axis_name="s",
                                  num_cores=_NC, num_subcores=_NS)
    n_out = NR * _R
    BND = ((NR + 1 + 16 + 7) // 8) * 8

    def body(*refs):
        if gather:
            (table, gidx, lidx, bounds, out, acc, gidx_v, lidx_v, rows_v,
             bounds_v, sem) = refs
        else:
            (lidx, bounds, out, acc, lidx_v, bounds_v, sem) = refs
            rows_v = None
        cid = lax.axis_index("c")
        sid = lax.axis_index("s")
        wid = cid * _NS + sid
        pltpu.sync_copy(bounds, bounds_v)

        def bval(i):
            return bounds_v[pl.ds(i, 16)][0]
        zero = jnp.zeros((16,), jnp.float32)
        one = jnp.ones((16,), jnp.float32)
        iota16 = lax.iota(jnp.int32, 16)

        nmine = (NR - wid + _NW - 1) // _NW

        def range_body(t, c):
            r = wid + t * _NW
            lo = bval(r)
            hi = bval(r + 1)

            def zi(i, c2):
                acc[pl.ds(i * 16, 16)] = zero
                return c2
            lax.fori_loop(0, _RZ * D // 16, zi, 0)

            lo128 = pl.multiple_of((lo // _C) * _C, _C)
            nch = (hi - lo128 + _C - 1) // _C

            def chunk(j, c2):
                s = pl.multiple_of(lo128 + j * _C, _C)
                pltpu.sync_copy(lidx.at[pl.ds(s, _C)], lidx_v)
                if gather:
                    pltpu.sync_copy(gidx.at[pl.ds(s, _C)], gidx_v)
                for q in range(_C // 16):
                    pos = s + q * 16 + iota16
                    lv = lidx_v[pl.ds(q * 16, 16)]
                    bad = (pos < lo) | (pos >= hi)
                    lidx_v[pl.ds(q * 16, 16)] = jnp.where(bad, _R, lv)
                if gather:
                    pltpu.async_copy(table.at[gidx_v], rows_v, sem).wait()

                def edge(e, c3):
                    ev = jnp.full((16,), e, jnp.int32)
                    rowv = plsc.load_gather(lidx_v, [ev])
                    base = rowv * D
                    if gather:
                        for d in range(D // 16):
                            cols = d * 16 + iota16
                            vals = plsc.load_gather(rows_v, [ev, cols])
                            plsc.addupdate_scatter(acc, [base + cols], vals)
                    else:
                        plsc.addupdate_scatter(acc, [base + iota16], one)
                    return c3
                lax.fori_loop(0, _C, edge, 0)
                return c2
            lax.fori_loop(0, nch, chunk, 0)

            dsto = pl.multiple_of(r * _R * D, 8)
            pltpu.sync_copy(acc.at[pl.ds(0, _R * D)], out.at[pl.ds(dsto, _R * D)])
            return c
        lax.fori_loop(0, nmine, range_body, 0)

    scratch = [
        pltpu.VMEM((_RZ * D,), jnp.float32),
    ]
    if gather:
        scratch += [
            pltpu.VMEM((_C,), jnp.int32),
            pltpu.VMEM((_C,), jnp.int32),
            pltpu.VMEM((_C, D), jnp.float32),
        ]
    else:
        scratch += [
            pltpu.VMEM((_C,), jnp.int32),
        ]
    scratch += [
        pltpu.VMEM((BND,), jnp.int32),
        pltpu.SemaphoreType.DMA,
    ]
    return pl.kernel(
        body,
        out_type=jax.ShapeDtypeStruct((n_out * D,), jnp.float32),
        mesh=mesh,
        scratch_types=scratch,
        compiler_params=pltpu.CompilerParams(needs_layout_passes=False),
    )


_NRQ = N_QUBITS // _R
_NRG = N_GATES // _R


def _edge_plan(sort_key, other, NR):
    order = jnp.argsort(sort_key)
    key_s = jnp.take(sort_key, order)
    gidx = jnp.take(other, order)
    lidx = key_s % _R
    bnd = ((NR + 1 + 16 + 7) // 8) * 8
    bounds = jnp.searchsorted(
        key_s, jnp.arange(NR + 1, dtype=jnp.int32) * _R).astype(jnp.int32)
    gidx_p = jnp.zeros((E_PAD,), jnp.int32).at[:N_EDGES].set(gidx)
    lidx_p = jnp.full((E_PAD,), _R, jnp.int32).at[:N_EDGES].set(lidx)
    bounds_p = jnp.full((bnd,), N_EDGES, jnp.int32).at[:NR + 1].set(bounds)
    return gidx_p, lidx_p, bounds_p


def kernel(gate_type_idx, gate_arity, gate_index_norm, gate_is_directional, qubit_degree_norm, edge_src_gate, edge_dst_qubit, emb_table, W_gate_in, b_gate_in, W_qubit_in, b_qubit_in, Wl_gq_0, bl_gq_0, Wr_gq_0, Wl_qg_0, bl_qg_0, Wr_qg_0, ln_g_0_s, ln_g_0_b, ln_q_0_s, ln_q_0_b, Wl_gq_1, bl_gq_1, Wr_gq_1, Wl_qg_1, bl_qg_1, Wr_qg_1, ln_g_1_s, ln_g_1_b, ln_q_1_s, ln_q_1_b, Wl_gq_2, bl_gq_2, Wr_gq_2, Wl_qg_2, bl_qg_2, Wr_qg_2, ln_g_2_s, ln_g_2_b, ln_q_2_s, ln_q_2_b, Wl_gq_3, bl_gq_3, Wr_gq_3, Wl_qg_3, bl_qg_3, Wr_qg_3, ln_g_3_s, ln_g_3_b, ln_q_3_s, ln_q_3_b, W_head1, b_head1, W_head2, b_head2):
    Wl = [Wl_gq_0, Wl_qg_0, Wl_gq_1, Wl_qg_1, Wl_gq_2, Wl_qg_2, Wl_gq_3, Wl_qg_3]
    bl = [bl_gq_0, bl_qg_0, bl_gq_1, bl_qg_1, bl_gq_2, bl_qg_2, bl_gq_3, bl_qg_3]
    Wr = [Wr_gq_0, Wr_qg_0, Wr_gq_1, Wr_qg_1, Wr_gq_2, Wr_qg_2, Wr_gq_3, Wr_qg_3]
    ln_s = [ln_g_0_s, ln_q_0_s, ln_g_1_s, ln_q_1_s, ln_g_2_s, ln_q_2_s,
            ln_g_3_s, ln_q_3_s]
    ln_b = [ln_g_0_b, ln_q_0_b, ln_g_1_b, ln_q_1_b, ln_g_2_b, ln_q_2_b,
            ln_g_3_b, ln_q_3_b]

    k1, k2, k3, k4 = jax.random.split(jax.random.key(42), 4)
    rand = jax.random.uniform(k1, (N_GATES,))
    mask_pos = rand < MASK_RATIO
    mask_rand = jax.random.uniform(k2, (N_GATES,))
    masked_type = jnp.where(mask_pos & (mask_rand < 0.8), MASK_TOKEN_IDX,
                            gate_type_idx)
    rnd_gates = jax.random.randint(k3, (N_GATES,), 0, NUM_GATE_TYPES)
    masked_type = jnp.where(mask_pos & (mask_rand >= 0.8) & (mask_rand < 0.9),
                            rnd_gates, masked_type)
    masked_arity = jnp.where(mask_pos, 0.0, gate_arity.astype(jnp.float32))
    masked_posf = jnp.where(mask_pos, 0.0, gate_index_norm)
    q_mask = jax.random.uniform(k4, (N_QUBITS,)) < QUBIT_MASK_RATIO
    masked_deg = jnp.where(q_mask, 0.0, qubit_degree_norm)

    gate_x = _gate_input_proj(masked_type.astype(jnp.int32), masked_arity,
                              gate_is_directional.astype(jnp.float32),
                              masked_posf, emb_table, W_gate_in, b_gate_in)
    qubit_x = _qubit_input_proj(masked_deg, W_qubit_in, b_qubit_in)

    src = edge_src_gate.astype(jnp.int32)
    dst = edge_dst_qubit.astype(jnp.int32)
    gidx_q, lidx_q, bounds_q = _edge_plan(dst, src, _NRQ)
    gidx_g, lidx_g, bounds_g = _edge_plan(src, dst, _NRG)

    agg_q_k = _sc_segsum(_NRQ, HIDDEN, True)
    agg_g_k = _sc_segsum(_NRG, HIDDEN, True)
    deg_q_k = _sc_segsum(_NRQ, 16, False)
    deg_g_k = _sc_segsum(_NRG, 16, False)

    deg_q = deg_q_k(lidx_q, bounds_q).reshape(N_QUBITS, 16)[:, :1]
    deg_g = deg_g_k(lidx_g, bounds_g).reshape(N_GATES, 16)[:, :1]

    for l in range(NUM_LAYERS):
        agg_q = agg_q_k(gate_x, gidx_q, lidx_q, bounds_q).reshape(
            N_QUBITS, HIDDEN)
        agg_g = agg_g_k(qubit_x, gidx_g, lidx_g, bounds_g).reshape(
            N_GATES, HIDDEN)
        new_q = _dense_update(agg_q, deg_q, qubit_x, Wl[2 * l], bl[2 * l],
                              Wr[2 * l], ln_s[2 * l + 1], ln_b[2 * l + 1], QB)
        new_g = _dense_update(agg_g, deg_g, gate_x, Wl[2 * l + 1], bl[2 * l + 1],
                              Wr[2 * l + 1], ln_s[2 * l], ln_b[2 * l], GB)
        gate_x, qubit_x = new_g, new_q

    return _head(gate_x, W_head1, b_head1, W_head2, b_head2)


# speedup vs baseline: 1.8717x; 1.4061x over previous
import functools

import jax
import jax.numpy as jnp
from jax import lax
from jax.experimental import pallas as pl
from jax.experimental.pallas import tpu as pltpu
from jax.experimental.pallas import tpu_sc as plsc

NUM_GATE_TYPES = 30
MASK_TOKEN_IDX = NUM_GATE_TYPES + 1
HIDDEN = 256
NUM_LAYERS = 4
GATE_EMB_DIM = 16
MASK_RATIO = 0.15
QUBIT_MASK_RATIO = 0.15
N_GATES = 100000
N_QUBITS = 10000
N_EDGES = 200000

GB = 2000
QB = 2000


def _silu(x):
    return x / (1.0 + jnp.exp(-x))


def _ln(x, s, b):
    mu = jnp.mean(x, axis=-1, keepdims=True)
    xc = x - mu
    var = jnp.mean(xc * xc, axis=-1, keepdims=True)
    return xc * jax.lax.rsqrt(var + 1e-5) * s + b


def _input_body(mtype_ref, arity_ref, isdir_ref, posf_ref, emb_ref, w16_ref,
                wadp_ref, b_ref, out_ref):
    mtype = mtype_ref[...]
    oh = (mtype == lax.broadcasted_iota(jnp.int32, (1, NUM_GATE_TYPES + 2), 1)
          ).astype(jnp.float32)
    M = lax.dot_general(emb_ref[...], w16_ref[...], (((1,), (1,)), ((), ())),
                        preferred_element_type=jnp.float32)
    x = lax.dot_general(oh, M, (((1,), (0,)), ((), ())),
                        preferred_element_type=jnp.float32)
    wadp = wadp_ref[...]
    x = x + arity_ref[...] * wadp[0:1, :]
    x = x + isdir_ref[...] * wadp[1:2, :]
    x = x + posf_ref[...] * wadp[2:3, :]
    out_ref[...] = x + b_ref[...]


def _gate_input_proj(mtype, arity, isdir, posf, emb_table, W_gate_in, b_gate_in):
    n = N_GATES
    w16 = W_gate_in[:, :GATE_EMB_DIM]
    wadp = jnp.transpose(W_gate_in[:, GATE_EMB_DIM:])
    grid = (n // GB,)
    col = lambda i: (i, 0)
    return pl.pallas_call(
        _input_body,
        grid=grid,
        in_specs=[
            pl.BlockSpec((GB, 1), col),
            pl.BlockSpec((GB, 1), col),
            pl.BlockSpec((GB, 1), col),
            pl.BlockSpec((GB, 1), col),
            pl.BlockSpec((NUM_GATE_TYPES + 2, GATE_EMB_DIM), lambda i: (0, 0)),
            pl.BlockSpec((HIDDEN, GATE_EMB_DIM), lambda i: (0, 0)),
            pl.BlockSpec((3, HIDDEN), lambda i: (0, 0)),
            pl.BlockSpec((1, HIDDEN), lambda i: (0, 0)),
        ],
        out_specs=pl.BlockSpec((GB, HIDDEN), col),
        out_shape=jax.ShapeDtypeStruct((n, HIDDEN), jnp.float32),
    )(mtype.reshape(n, 1), arity.reshape(n, 1), isdir.reshape(n, 1),
      posf.reshape(n, 1), emb_table, w16, wadp, b_gate_in.reshape(1, HIDDEN))


def _qubit_input_body(deg_ref, w_ref, b_ref, out_ref):
    out_ref[...] = deg_ref[...] * w_ref[...] + b_ref[...]


def _qubit_input_proj(masked_deg, W_qubit_in, b_qubit_in):
    n = N_QUBITS
    return pl.pallas_call(
        _qubit_input_body,
        grid=(n // QB,),
        in_specs=[
            pl.BlockSpec((QB, 1), lambda i: (i, 0)),
            pl.BlockSpec((1, HIDDEN), lambda i: (0, 0)),
            pl.BlockSpec((1, HIDDEN), lambda i: (0, 0)),
        ],
        out_specs=pl.BlockSpec((QB, HIDDEN), lambda i: (i, 0)),
        out_shape=jax.ShapeDtypeStruct((n, HIDDEN), jnp.float32),
    )(masked_deg.reshape(n, 1), W_qubit_in.reshape(1, HIDDEN),
      b_qubit_in.reshape(1, HIDDEN))


def _dense_body(agg_ref, deg_ref, x_ref, wl_ref, bl_ref, wr_ref, s_ref, b_ref,
                out_ref):
    x = x_ref[...]
    agg = agg_ref[...] / jnp.maximum(deg_ref[...], 1.0)
    h = lax.dot_general(agg, wl_ref[...], (((1,), (1,)), ((), ())),
                        preferred_element_type=jnp.float32)
    h = h + lax.dot_general(x, wr_ref[...], (((1,), (1,)), ((), ())),
                            preferred_element_type=jnp.float32)
    h = h + bl_ref[...]
    out_ref[...] = _ln(_silu(h), s_ref[...], b_ref[...]) + x


def _dense_update(agg, deg, x, Wl, bl, Wr, ln_s, ln_b, blk):
    n = x.shape[0]
    full = lambda i: (0, 0)
    col = lambda i: (i, 0)
    return pl.pallas_call(
        _dense_body,
        grid=(n // blk,),
        in_specs=[
            pl.BlockSpec((blk, HIDDEN), col),
            pl.BlockSpec((blk, 1), col),
            pl.BlockSpec((blk, HIDDEN), col),
            pl.BlockSpec((HIDDEN, HIDDEN), full),
            pl.BlockSpec((1, HIDDEN), full),
            pl.BlockSpec((HIDDEN, HIDDEN), full),
            pl.BlockSpec((1, HIDDEN), full),
            pl.BlockSpec((1, HIDDEN), full),
        ],
        out_specs=pl.BlockSpec((blk, HIDDEN), col),
        out_shape=jax.ShapeDtypeStruct((n, HIDDEN), jnp.float32),
    )(agg, deg.reshape(n, 1), x, Wl, bl.reshape(1, HIDDEN), Wr,
      ln_s.reshape(1, HIDDEN), ln_b.reshape(1, HIDDEN))


def _head_body(x_ref, w1_ref, b1_ref, w2_ref, b2_ref, out_ref):
    h = lax.dot_general(x_ref[...], w1_ref[...], (((1,), (1,)), ((), ())),
                        preferred_element_type=jnp.float32)
    h = _silu(h + b1_ref[...])
    out_ref[...] = lax.dot_general(h, w2_ref[...], (((1,), (1,)), ((), ())),
                                   preferred_element_type=jnp.float32) + b2_ref[...]


def _head(gate_x, W_head1, b_head1, W_head2, b_head2):
    n = N_GATES
    nt = NUM_GATE_TYPES + 1
    full = lambda i: (0, 0)
    return pl.pallas_call(
        _head_body,
        grid=(n // GB,),
        in_specs=[
            pl.BlockSpec((GB, HIDDEN), lambda i: (i, 0)),
            pl.BlockSpec((HIDDEN, HIDDEN), full),
            pl.BlockSpec((1, HIDDEN), full),
            pl.BlockSpec((nt, HIDDEN), full),
            pl.BlockSpec((1, nt), full),
        ],
        out_specs=pl.BlockSpec((GB, nt), lambda i: (i, 0)),
        out_shape=jax.ShapeDtypeStruct((n, nt), jnp.float32),
    )(gate_x, W_head1, b_head1.reshape(1, HIDDEN), W_head2,
      b_head2.reshape(1, nt))


_NC, _NS = 2, 16
_NW = _NC * _NS
_C = 128
_R = 200
_RZ = 208
E_PAD = N_EDGES + 448


def _sc_segsum(NR, D, gather):
    mesh = plsc.VectorSubcoreMesh(core_axis_name="c", subcore_axis_name="s",
                                  num_cores=_NC, num_subcores=_NS)
    n_out = NR * _R
    BND = ((NR + 1 + 16 + 7) // 8) * 8

    def body(*refs):
        if gather:
            (table, ipair, bounds, out, acc, iv0, iv1, rows0, rows1,
             bounds_v, si0, si1, sg0, sg1) = refs
            ivs, rows, sis, sgs = (iv0, iv1), (rows0, rows1), (si0, si1), \
                (sg0, sg1)
        else:
            (ipair, bounds, out, acc, iv0, iv1, bounds_v, si0, si1) = refs
            ivs, sis = (iv0, iv1), (si0, si1)
            rows = sgs = None
        cid = lax.axis_index("c")
        sid = lax.axis_index("s")
        wid = cid * _NS + sid
        pltpu.sync_copy(bounds, bounds_v)

        def bval(i):
            return bounds_v[pl.ds(i, 16)][0]
        zero = jnp.zeros((16,), jnp.float32)
        one = jnp.ones((16,), jnp.float32)
        iota16 = lax.iota(jnp.int32, 16)
        IW = 2 * _C if gather else _C

        nmine = (NR - wid + _NW - 1) // _NW

        def range_body(t, c):
            r = wid + t * _NW
            lo = bval(r)
            hi = bval(r + 1)
            lo128 = pl.multiple_of((lo // _C) * _C, _C)
            nch = (hi - lo128 + _C - 1) // _C

            def start_idx(j, b):
                off = pl.multiple_of((lo128 // _C + j) * IW, 8)
                pltpu.async_copy(ipair.at[pl.ds(off, IW)], ivs[b], sis[b])

            def wait_idx(b):
                pltpu.make_async_copy(ipair.at[pl.ds(0, IW)], ivs[b],
                                      sis[b]).wait()

            def start_gather(b):
                pltpu.async_copy(table.at[ivs[b].at[pl.ds(0, _C)]], rows[b],
                                 sgs[b])

            def wait_gather(b):
                pltpu.make_async_copy(table.at[ivs[b].at[pl.ds(0, _C)]],
                                      rows[b], sgs[b]).wait()

            pl.when(nch > 0)(lambda: start_idx(0, 0))
            pl.when(nch > 1)(lambda: start_idx(1, 1))

            def zi(i, c2):
                acc[pl.ds(i * 16, 16)] = zero
                return c2
            lax.fori_loop(0, _RZ * D // 16, zi, 0, unroll=8)

            if gather:
                def prol():
                    wait_idx(0)
                    start_gather(0)
                pl.when(nch > 0)(prol)
            else:
                pl.when(nch > 0)(lambda: wait_idx(0))

            loff = _C if gather else 0

            def process(j, b):
                s = pl.multiple_of(lo128 + j * _C, _C)
                iv = ivs[b]
                if gather:
                    wait_gather(b)
                for q in range(_C // 16):
                    pos = s + q * 16 + iota16
                    lv = iv[pl.ds(loff + q * 16, 16)]
                    bad = (pos < lo) | (pos >= hi)
                    iv[pl.ds(loff + q * 16, 16)] = jnp.where(bad, _R, lv)

                def edge(e, c3):
                    ev = jnp.full((16,), loff + e, jnp.int32)
                    rowv = plsc.load_gather(iv, [ev])
                    base = rowv * D
                    if gather:
                        evr = jnp.full((16,), e, jnp.int32)
                        for d in range(D // 16):
                            cols = d * 16 + iota16
                            vals = plsc.load_gather(rows[b], [evr, cols])
                            plsc.addupdate_scatter(acc, [base + cols], vals)
                    else:
                        plsc.addupdate_scatter(acc, [base + iota16], one)
                    return c3
                lax.fori_loop(0, _C, edge, 0, unroll=4)

            def chunk2(jj, c2):
                for u in range(2):
                    j = jj * 2 + u

                    def step(j=j, b=u):
                        bn = 1 - b
                        if gather:
                            def pref():
                                wait_idx(bn)
                                start_gather(bn)
                            pl.when(j + 1 < nch)(pref)
                        else:
                            pl.when(j + 1 < nch)(lambda: wait_idx(bn))
                        process(j, b)
                        pl.when(j + 2 < nch)(lambda: start_idx(j + 2, b))
                    pl.when(j < nch)(step)
                return c2
            lax.fori_loop(0, (nch + 1) // 2, chunk2, 0)

            dsto = pl.multiple_of(r * _R * D, 8)
            pltpu.sync_copy(acc.at[pl.ds(0, _R * D)], out.at[pl.ds(dsto, _R * D)])
            return c
        lax.fori_loop(0, nmine, range_body, 0)

    scratch = [
        pltpu.VMEM((_RZ * D,), jnp.float32),
        pltpu.VMEM((2 * _C if gather else _C,), jnp.int32),
        pltpu.VMEM((2 * _C if gather else _C,), jnp.int32),
    ]
    if gather:
        scratch += [
            pltpu.VMEM((_C, D), jnp.float32),
            pltpu.VMEM((_C, D), jnp.float32),
        ]
    scratch += [
        pltpu.VMEM((BND,), jnp.int32),
        pltpu.SemaphoreType.DMA,
        pltpu.SemaphoreType.DMA,
    ]
    if gather:
        scratch += [
            pltpu.SemaphoreType.DMA,
            pltpu.SemaphoreType.DMA,
        ]
    return pl.kernel(
        body,
        out_type=jax.ShapeDtypeStruct((n_out * D,), jnp.float32),
        mesh=mesh,
        scratch_types=scratch,
        compiler_params=pltpu.CompilerParams(needs_layout_passes=False),
    )


_NRQ = N_QUBITS // _R
_NRG = N_GATES // _R


def _edge_plan(sort_key, other, NR):
    order = jnp.argsort(sort_key)
    key_s = jnp.take(sort_key, order)
    gidx = jnp.take(other, order)
    lidx = key_s % _R
    bnd = ((NR + 1 + 16 + 7) // 8) * 8
    bounds = jnp.searchsorted(
        key_s, jnp.arange(NR + 1, dtype=jnp.int32) * _R).astype(jnp.int32)
    gidx_p = jnp.zeros((E_PAD,), jnp.int32).at[:N_EDGES].set(gidx)
    lidx_p = jnp.full((E_PAD,), _R, jnp.int32).at[:N_EDGES].set(lidx)
    bounds_p = jnp.full((bnd,), N_EDGES, jnp.int32).at[:NR + 1].set(bounds)
    ipair = jnp.stack([gidx_p.reshape(-1, _C), lidx_p.reshape(-1, _C)],
                      axis=1).reshape(-1)
    return ipair, lidx_p, bounds_p


def kernel(gate_type_idx, gate_arity, gate_index_norm, gate_is_directional, qubit_degree_norm, edge_src_gate, edge_dst_qubit, emb_table, W_gate_in, b_gate_in, W_qubit_in, b_qubit_in, Wl_gq_0, bl_gq_0, Wr_gq_0, Wl_qg_0, bl_qg_0, Wr_qg_0, ln_g_0_s, ln_g_0_b, ln_q_0_s, ln_q_0_b, Wl_gq_1, bl_gq_1, Wr_gq_1, Wl_qg_1, bl_qg_1, Wr_qg_1, ln_g_1_s, ln_g_1_b, ln_q_1_s, ln_q_1_b, Wl_gq_2, bl_gq_2, Wr_gq_2, Wl_qg_2, bl_qg_2, Wr_qg_2, ln_g_2_s, ln_g_2_b, ln_q_2_s, ln_q_2_b, Wl_gq_3, bl_gq_3, Wr_gq_3, Wl_qg_3, bl_qg_3, Wr_qg_3, ln_g_3_s, ln_g_3_b, ln_q_3_s, ln_q_3_b, W_head1, b_head1, W_head2, b_head2):
    Wl = [Wl_gq_0, Wl_qg_0, Wl_gq_1, Wl_qg_1, Wl_gq_2, Wl_qg_2, Wl_gq_3, Wl_qg_3]
    bl = [bl_gq_0, bl_qg_0, bl_gq_1, bl_qg_1, bl_gq_2, bl_qg_2, bl_gq_3, bl_qg_3]
    Wr = [Wr_gq_0, Wr_qg_0, Wr_gq_1, Wr_qg_1, Wr_gq_2, Wr_qg_2, Wr_gq_3, Wr_qg_3]
    ln_s = [ln_g_0_s, ln_q_0_s, ln_g_1_s, ln_q_1_s, ln_g_2_s, ln_q_2_s,
            ln_g_3_s, ln_q_3_s]
    ln_b = [ln_g_0_b, ln_q_0_b, ln_g_1_b, ln_q_1_b, ln_g_2_b, ln_q_2_b,
            ln_g_3_b, ln_q_3_b]

    k1, k2, k3, k4 = jax.random.split(jax.random.key(42), 4)
    rand = jax.random.uniform(k1, (N_GATES,))
    mask_pos = rand < MASK_RATIO
    mask_rand = jax.random.uniform(k2, (N_GATES,))
    masked_type = jnp.where(mask_pos & (mask_rand < 0.8), MASK_TOKEN_IDX,
                            gate_type_idx)
    rnd_gates = jax.random.randint(k3, (N_GATES,), 0, NUM_GATE_TYPES)
    masked_type = jnp.where(mask_pos & (mask_rand >= 0.8) & (mask_rand < 0.9),
                            rnd_gates, masked_type)
    masked_arity = jnp.where(mask_pos, 0.0, gate_arity.astype(jnp.float32))
    masked_posf = jnp.where(mask_pos, 0.0, gate_index_norm)
    q_mask = jax.random.uniform(k4, (N_QUBITS,)) < QUBIT_MASK_RATIO
    masked_deg = jnp.where(q_mask, 0.0, qubit_degree_norm)

    gate_x = _gate_input_proj(masked_type.astype(jnp.int32), masked_arity,
                              gate_is_directional.astype(jnp.float32),
                              masked_posf, emb_table, W_gate_in, b_gate_in)
    qubit_x = _qubit_input_proj(masked_deg, W_qubit_in, b_qubit_in)

    src = edge_src_gate.astype(jnp.int32)
    dst = edge_dst_qubit.astype(jnp.int32)
    ipair_q, lidx_q, bounds_q = _edge_plan(dst, src, _NRQ)
    ipair_g, lidx_g, bounds_g = _edge_plan(src, dst, _NRG)

    agg_q_k = _sc_segsum(_NRQ, HIDDEN, True)
    agg_g_k = _sc_segsum(_NRG, HIDDEN, True)
    deg_q_k = _sc_segsum(_NRQ, 16, False)
    deg_g_k = _sc_segsum(_NRG, 16, False)

    deg_q = deg_q_k(lidx_q, bounds_q).reshape(N_QUBITS, 16)[:, :1]
    deg_g = deg_g_k(lidx_g, bounds_g).reshape(N_GATES, 16)[:, :1]

    for l in range(NUM_LAYERS):
        agg_q = agg_q_k(gate_x, ipair_q, bounds_q).reshape(N_QUBITS, HIDDEN)
        agg_g = agg_g_k(qubit_x, ipair_g, bounds_g).reshape(N_GATES, HIDDEN)
        new_q = _dense_update(agg_q, deg_q, qubit_x, Wl[2 * l], bl[2 * l],
                              Wr[2 * l], ln_s[2 * l + 1], ln_b[2 * l + 1], QB)
        new_g = _dense_update(agg_g, deg_g, gate_x, Wl[2 * l + 1], bl[2 * l + 1],
                              Wr[2 * l + 1], ln_s[2 * l], ln_b[2 * l], GB)
        gate_x, qubit_x = new_g, new_q

    return _head(gate_x, W_head1, b_head1, W_head2, b_head2)


# speedup vs baseline: 3.2560x; 1.7396x over previous
import functools

import jax
import jax.numpy as jnp
from jax import lax
from jax.experimental import pallas as pl
from jax.experimental.pallas import tpu as pltpu
from jax.experimental.pallas import tpu_sc as plsc

NUM_GATE_TYPES = 30
MASK_TOKEN_IDX = NUM_GATE_TYPES + 1
HIDDEN = 256
NUM_LAYERS = 4
GATE_EMB_DIM = 16
MASK_RATIO = 0.15
QUBIT_MASK_RATIO = 0.15
N_GATES = 100000
N_QUBITS = 10000
N_EDGES = 200000

GB = 2000
QB = 2000


def _silu(x):
    return x / (1.0 + jnp.exp(-x))


def _ln(x, s, b):
    mu = jnp.mean(x, axis=-1, keepdims=True)
    xc = x - mu
    var = jnp.mean(xc * xc, axis=-1, keepdims=True)
    return xc * jax.lax.rsqrt(var + 1e-5) * s + b


def _input_body(mtype_ref, arity_ref, isdir_ref, posf_ref, emb_ref, w16_ref,
                wadp_ref, b_ref, out_ref):
    mtype = mtype_ref[...]
    oh = (mtype == lax.broadcasted_iota(jnp.int32, (1, NUM_GATE_TYPES + 2), 1)
          ).astype(jnp.float32)
    M = lax.dot_general(emb_ref[...], w16_ref[...], (((1,), (1,)), ((), ())),
                        preferred_element_type=jnp.float32)
    x = lax.dot_general(oh, M, (((1,), (0,)), ((), ())),
                        preferred_element_type=jnp.float32)
    wadp = wadp_ref[...]
    x = x + arity_ref[...] * wadp[0:1, :]
    x = x + isdir_ref[...] * wadp[1:2, :]
    x = x + posf_ref[...] * wadp[2:3, :]
    out_ref[...] = x + b_ref[...]


def _gate_input_proj(mtype, arity, isdir, posf, emb_table, W_gate_in, b_gate_in):
    n = N_GATES
    w16 = W_gate_in[:, :GATE_EMB_DIM]
    wadp = jnp.transpose(W_gate_in[:, GATE_EMB_DIM:])
    grid = (n // GB,)
    col = lambda i: (i, 0)
    return pl.pallas_call(
        _input_body,
        grid=grid,
        in_specs=[
            pl.BlockSpec((GB, 1), col),
            pl.BlockSpec((GB, 1), col),
            pl.BlockSpec((GB, 1), col),
            pl.BlockSpec((GB, 1), col),
            pl.BlockSpec((NUM_GATE_TYPES + 2, GATE_EMB_DIM), lambda i: (0, 0)),
            pl.BlockSpec((HIDDEN, GATE_EMB_DIM), lambda i: (0, 0)),
            pl.BlockSpec((3, HIDDEN), lambda i: (0, 0)),
            pl.BlockSpec((1, HIDDEN), lambda i: (0, 0)),
        ],
        out_specs=pl.BlockSpec((GB, HIDDEN), col),
        out_shape=jax.ShapeDtypeStruct((n, HIDDEN), jnp.float32),
    )(mtype.reshape(n, 1), arity.reshape(n, 1), isdir.reshape(n, 1),
      posf.reshape(n, 1), emb_table, w16, wadp, b_gate_in.reshape(1, HIDDEN))


def _qubit_input_body(deg_ref, w_ref, b_ref, out_ref):
    out_ref[...] = deg_ref[...] * w_ref[...] + b_ref[...]


def _qubit_input_proj(masked_deg, W_qubit_in, b_qubit_in):
    n = N_QUBITS
    return pl.pallas_call(
        _qubit_input_body,
        grid=(n // QB,),
        in_specs=[
            pl.BlockSpec((QB, 1), lambda i: (i, 0)),
            pl.BlockSpec((1, HIDDEN), lambda i: (0, 0)),
            pl.BlockSpec((1, HIDDEN), lambda i: (0, 0)),
        ],
        out_specs=pl.BlockSpec((QB, HIDDEN), lambda i: (i, 0)),
        out_shape=jax.ShapeDtypeStruct((n, HIDDEN), jnp.float32),
    )(masked_deg.reshape(n, 1), W_qubit_in.reshape(1, HIDDEN),
      b_qubit_in.reshape(1, HIDDEN))


def _dense_body(agg_ref, deg_ref, x_ref, wl_ref, bl_ref, wr_ref, s_ref, b_ref,
                out_ref):
    x = x_ref[...]
    agg = agg_ref[...] / jnp.maximum(deg_ref[...], 1.0)
    h = lax.dot_general(agg, wl_ref[...], (((1,), (1,)), ((), ())),
                        preferred_element_type=jnp.float32)
    h = h + lax.dot_general(x, wr_ref[...], (((1,), (1,)), ((), ())),
                            preferred_element_type=jnp.float32)
    h = h + bl_ref[...]
    out_ref[...] = _ln(_silu(h), s_ref[...], b_ref[...]) + x


def _dense_update(agg, deg, x, Wl, bl, Wr, ln_s, ln_b, blk):
    n = x.shape[0]
    full = lambda i: (0, 0)
    col = lambda i: (i, 0)
    return pl.pallas_call(
        _dense_body,
        grid=(n // blk,),
        in_specs=[
            pl.BlockSpec((blk, HIDDEN), col),
            pl.BlockSpec((blk, 1), col),
            pl.BlockSpec((blk, HIDDEN), col),
            pl.BlockSpec((HIDDEN, HIDDEN), full),
            pl.BlockSpec((1, HIDDEN), full),
            pl.BlockSpec((HIDDEN, HIDDEN), full),
            pl.BlockSpec((1, HIDDEN), full),
            pl.BlockSpec((1, HIDDEN), full),
        ],
        out_specs=pl.BlockSpec((blk, HIDDEN), col),
        out_shape=jax.ShapeDtypeStruct((n, HIDDEN), jnp.float32),
    )(agg, deg.reshape(n, 1), x, Wl, bl.reshape(1, HIDDEN), Wr,
      ln_s.reshape(1, HIDDEN), ln_b.reshape(1, HIDDEN))


def _head_body(x_ref, w1_ref, b1_ref, w2_ref, b2_ref, out_ref):
    h = lax.dot_general(x_ref[...], w1_ref[...], (((1,), (1,)), ((), ())),
                        preferred_element_type=jnp.float32)
    h = _silu(h + b1_ref[...])
    out_ref[...] = lax.dot_general(h, w2_ref[...], (((1,), (1,)), ((), ())),
                                   preferred_element_type=jnp.float32) + b2_ref[...]


def _head(gate_x, W_head1, b_head1, W_head2, b_head2):
    n = N_GATES
    nt = NUM_GATE_TYPES + 1
    full = lambda i: (0, 0)
    return pl.pallas_call(
        _head_body,
        grid=(n // GB,),
        in_specs=[
            pl.BlockSpec((GB, HIDDEN), lambda i: (i, 0)),
            pl.BlockSpec((HIDDEN, HIDDEN), full),
            pl.BlockSpec((1, HIDDEN), full),
            pl.BlockSpec((nt, HIDDEN), full),
            pl.BlockSpec((1, nt), full),
        ],
        out_specs=pl.BlockSpec((GB, nt), lambda i: (i, 0)),
        out_shape=jax.ShapeDtypeStruct((n, nt), jnp.float32),
    )(gate_x, W_head1, b_head1.reshape(1, HIDDEN), W_head2,
      b_head2.reshape(1, nt))


_NC, _NS = 2, 16
_NW = _NC * _NS
_C = 128
_R = 200
_RZ = 208
E_PAD = N_EDGES + 448


def _sc_segsum(NR, D, gather):
    mesh = plsc.VectorSubcoreMesh(core_axis_name="c", subcore_axis_name="s",
                                  num_cores=_NC, num_subcores=_NS)
    n_out = NR * _R
    BND = ((NR + 1 + 16 + 7) // 8) * 8

    def body(*refs):
        if gather:
            (table, ipair, bounds, out, acc, iv0, iv1, rows0, rows1,
             bounds_v, si0, si1, sg0, sg1) = refs
            ivs, rows, sis, sgs = (iv0, iv1), (rows0, rows1), (si0, si1), \
                (sg0, sg1)
        else:
            (ipair, bounds, out, acc, iv0, iv1, bounds_v, si0, si1) = refs
            ivs, sis = (iv0, iv1), (si0, si1)
            rows = sgs = None
        cid = lax.axis_index("c")
        sid = lax.axis_index("s")
        wid = cid * _NS + sid
        pltpu.sync_copy(bounds, bounds_v)

        def bval(i):
            return bounds_v[pl.ds(i, 16)][0]
        zero = jnp.zeros((16,), jnp.float32)
        one = jnp.ones((16,), jnp.float32)
        iota16 = lax.iota(jnp.int32, 16)
        IW = 2 * _C if gather else _C

        nmine = (NR - wid + _NW - 1) // _NW

        def range_body(t, c):
            r = wid + t * _NW
            lo = bval(r)
            hi = bval(r + 1)
            lo128 = pl.multiple_of((lo // _C) * _C, _C)
            nch = (hi - lo128 + _C - 1) // _C

            def start_idx(j, b):
                off = pl.multiple_of((lo128 // _C + j) * IW, 8)
                pltpu.async_copy(ipair.at[pl.ds(off, IW)], ivs[b], sis[b])

            def wait_idx(b):
                pltpu.make_async_copy(ipair.at[pl.ds(0, IW)], ivs[b],
                                      sis[b]).wait()

            def start_gather(b):
                pltpu.async_copy(table.at[ivs[b].at[pl.ds(0, _C)]], rows[b],
                                 sgs[b])

            def wait_gather(b):
                pltpu.make_async_copy(table.at[ivs[b].at[pl.ds(0, _C)]],
                                      rows[b], sgs[b]).wait()

            pl.when(nch > 0)(lambda: start_idx(0, 0))
            pl.when(nch > 1)(lambda: start_idx(1, 1))

            def zi(i, c2):
                acc[pl.ds(i * 16, 16)] = zero
                return c2
            lax.fori_loop(0, _RZ * D // 16, zi, 0, unroll=8)

            if gather:
                def prol():
                    wait_idx(0)
                    start_gather(0)
                pl.when(nch > 0)(prol)
            else:
                pl.when(nch > 0)(lambda: wait_idx(0))

            loff = _C if gather else 0

            def process(j, b):
                s = pl.multiple_of(lo128 + j * _C, _C)
                iv = ivs[b]
                if gather:
                    wait_gather(b)
                for q in range(_C // 16):
                    pos = s + q * 16 + iota16
                    lv = iv[pl.ds(loff + q * 16, 16)]
                    bad = (pos < lo) | (pos >= hi)
                    iv[pl.ds(loff + q * 16, 16)] = jnp.where(bad, _R, lv)

                @plsc.parallel_loop(0, _C, 1, unroll=4)
                def edge(e):
                    ev = jnp.full((16,), loff + e, jnp.int32)
                    rowv = plsc.load_gather(iv, [ev])
                    base = rowv * D
                    if gather:
                        evr = jnp.full((16,), e, jnp.int32)
                        for d in range(D // 16):
                            cols = d * 16 + iota16
                            vals = plsc.load_gather(rows[b], [evr, cols])
                            plsc.addupdate_scatter(acc, [base + cols], vals)
                    else:
                        plsc.addupdate_scatter(acc, [base + iota16], one)

            def chunk2(jj, c2):
                for u in range(2):
                    j = jj * 2 + u

                    def step(j=j, b=u):
                        bn = 1 - b
                        if gather:
                            def pref():
                                wait_idx(bn)
                                start_gather(bn)
                            pl.when(j + 1 < nch)(pref)
                        else:
                            pl.when(j + 1 < nch)(lambda: wait_idx(bn))
                        process(j, b)
                        pl.when(j + 2 < nch)(lambda: start_idx(j + 2, b))
                    pl.when(j < nch)(step)
                return c2
            lax.fori_loop(0, (nch + 1) // 2, chunk2, 0)

            dsto = pl.multiple_of(r * _R * D, 8)
            pltpu.sync_copy(acc.at[pl.ds(0, _R * D)], out.at[pl.ds(dsto, _R * D)])
            return c
        lax.fori_loop(0, nmine, range_body, 0)

    scratch = [
        pltpu.VMEM((_RZ * D,), jnp.float32),
        pltpu.VMEM((2 * _C if gather else _C,), jnp.int32),
        pltpu.VMEM((2 * _C if gather else _C,), jnp.int32),
    ]
    if gather:
        scratch += [
            pltpu.VMEM((_C, D), jnp.float32),
            pltpu.VMEM((_C, D), jnp.float32),
        ]
    scratch += [
        pltpu.VMEM((BND,), jnp.int32),
        pltpu.SemaphoreType.DMA,
        pltpu.SemaphoreType.DMA,
    ]
    if gather:
        scratch += [
            pltpu.SemaphoreType.DMA,
            pltpu.SemaphoreType.DMA,
        ]
    return pl.kernel(
        body,
        out_type=jax.ShapeDtypeStruct((n_out * D,), jnp.float32),
        mesh=mesh,
        scratch_types=scratch,
        compiler_params=pltpu.CompilerParams(needs_layout_passes=False),
    )


_NRQ = N_QUBITS // _R
_NRG = N_GATES // _R


def _edge_plan(sort_key, other, NR):
    order = jnp.argsort(sort_key)
    key_s = jnp.take(sort_key, order)
    gidx = jnp.take(other, order)
    lidx = key_s % _R
    bnd = ((NR + 1 + 16 + 7) // 8) * 8
    bounds = jnp.searchsorted(
        key_s, jnp.arange(NR + 1, dtype=jnp.int32) * _R).astype(jnp.int32)
    gidx_p = jnp.zeros((E_PAD,), jnp.int32).at[:N_EDGES].set(gidx)
    lidx_p = jnp.full((E_PAD,), _R, jnp.int32).at[:N_EDGES].set(lidx)
    bounds_p = jnp.full((bnd,), N_EDGES, jnp.int32).at[:NR + 1].set(bounds)
    ipair = jnp.stack([gidx_p.reshape(-1, _C), lidx_p.reshape(-1, _C)],
                      axis=1).reshape(-1)
    return ipair, lidx_p, bounds_p


def kernel(gate_type_idx, gate_arity, gate_index_norm, gate_is_directional, qubit_degree_norm, edge_src_gate, edge_dst_qubit, emb_table, W_gate_in, b_gate_in, W_qubit_in, b_qubit_in, Wl_gq_0, bl_gq_0, Wr_gq_0, Wl_qg_0, bl_qg_0, Wr_qg_0, ln_g_0_s, ln_g_0_b, ln_q_0_s, ln_q_0_b, Wl_gq_1, bl_gq_1, Wr_gq_1, Wl_qg_1, bl_qg_1, Wr_qg_1, ln_g_1_s, ln_g_1_b, ln_q_1_s, ln_q_1_b, Wl_gq_2, bl_gq_2, Wr_gq_2, Wl_qg_2, bl_qg_2, Wr_qg_2, ln_g_2_s, ln_g_2_b, ln_q_2_s, ln_q_2_b, Wl_gq_3, bl_gq_3, Wr_gq_3, Wl_qg_3, bl_qg_3, Wr_qg_3, ln_g_3_s, ln_g_3_b, ln_q_3_s, ln_q_3_b, W_head1, b_head1, W_head2, b_head2):
    Wl = [Wl_gq_0, Wl_qg_0, Wl_gq_1, Wl_qg_1, Wl_gq_2, Wl_qg_2, Wl_gq_3, Wl_qg_3]
    bl = [bl_gq_0, bl_qg_0, bl_gq_1, bl_qg_1, bl_gq_2, bl_qg_2, bl_gq_3, bl_qg_3]
    Wr = [Wr_gq_0, Wr_qg_0, Wr_gq_1, Wr_qg_1, Wr_gq_2, Wr_qg_2, Wr_gq_3, Wr_qg_3]
    ln_s = [ln_g_0_s, ln_q_0_s, ln_g_1_s, ln_q_1_s, ln_g_2_s, ln_q_2_s,
            ln_g_3_s, ln_q_3_s]
    ln_b = [ln_g_0_b, ln_q_0_b, ln_g_1_b, ln_q_1_b, ln_g_2_b, ln_q_2_b,
            ln_g_3_b, ln_q_3_b]

    k1, k2, k3, k4 = jax.random.split(jax.random.key(42), 4)
    rand = jax.random.uniform(k1, (N_GATES,))
    mask_pos = rand < MASK_RATIO
    mask_rand = jax.random.uniform(k2, (N_GATES,))
    masked_type = jnp.where(mask_pos & (mask_rand < 0.8), MASK_TOKEN_IDX,
                            gate_type_idx)
    rnd_gates = jax.random.randint(k3, (N_GATES,), 0, NUM_GATE_TYPES)
    masked_type = jnp.where(mask_pos & (mask_rand >= 0.8) & (mask_rand < 0.9),
                            rnd_gates, masked_type)
    masked_arity = jnp.where(mask_pos, 0.0, gate_arity.astype(jnp.float32))
    masked_posf = jnp.where(mask_pos, 0.0, gate_index_norm)
    q_mask = jax.random.uniform(k4, (N_QUBITS,)) < QUBIT_MASK_RATIO
    masked_deg = jnp.where(q_mask, 0.0, qubit_degree_norm)

    gate_x = _gate_input_proj(masked_type.astype(jnp.int32), masked_arity,
                              gate_is_directional.astype(jnp.float32),
                              masked_posf, emb_table, W_gate_in, b_gate_in)
    qubit_x = _qubit_input_proj(masked_deg, W_qubit_in, b_qubit_in)

    src = edge_src_gate.astype(jnp.int32)
    dst = edge_dst_qubit.astype(jnp.int32)
    ipair_q, lidx_q, bounds_q = _edge_plan(dst, src, _NRQ)
    ipair_g, lidx_g, bounds_g = _edge_plan(src, dst, _NRG)

    agg_q_k = _sc_segsum(_NRQ, HIDDEN, True)
    agg_g_k = _sc_segsum(_NRG, HIDDEN, True)
    deg_q_k = _sc_segsum(_NRQ, 16, False)
    deg_g_k = _sc_segsum(_NRG, 16, False)

    deg_q = deg_q_k(lidx_q, bounds_q).reshape(N_QUBITS, 16)[:, :1]
    deg_g = deg_g_k(lidx_g, bounds_g).reshape(N_GATES, 16)[:, :1]

    for l in range(NUM_LAYERS):
        agg_q = agg_q_k(gate_x, ipair_q, bounds_q).reshape(N_QUBITS, HIDDEN)
        agg_g = agg_g_k(qubit_x, ipair_g, bounds_g).reshape(N_GATES, HIDDEN)
        new_q = _dense_update(agg_q, deg_q, qubit_x, Wl[2 * l], bl[2 * l],
                              Wr[2 * l], ln_s[2 * l + 1], ln_b[2 * l + 1], QB)
        new_g = _dense_update(agg_g, deg_g, gate_x, Wl[2 * l + 1], bl[2 * l + 1],
                              Wr[2 * l + 1], ln_s[2 * l], ln_b[2 * l], GB)
        gate_x, qubit_x = new_g, new_q

    return _head(gate_x, W_head1, b_head1, W_head2, b_head2)
